# Initial kernel scaffold; baseline (speedup 1.0000x reference)
import jax
import jax.numpy as jnp
from jax.experimental import pallas as pl


def kernel(pooled_object_features, cls_score_object, concept_tokens, concept_lengths, valid_class_mask, Wi_f, Wh_f, bi_f, bh_f, Wi_b, Wh_b, bi_b, bh_b, Wq_w, Wq_b, W1_w, W1_b, W2_w, W2_b, Wr, Ur, br, Wc, Uc, bc, Wm_w, Wm_b, Wu_w, Wu_b):
    raise NotImplementedError("write your pallas kernel here")


# speedup vs baseline: 5.9406x; 5.9406x over previous
import functools

import jax
import jax.numpy as jnp
from jax import lax
from jax.experimental import pallas as pl
from jax.experimental.pallas import tpu as pltpu
from jax.experimental.pallas import tpu_sc as plsc

N_OBJ = 512
N_CLS = 151
TOP_K = 5
MAX_LEN = 16
D_W = 300
DP = 384
D_OBJ = 512
T_M = 3
ROWS = N_CLS * TOP_K
RP = 768
G3 = 3 * DP
D_TAB = TOP_K * DP + 128

SC_NC = 2
SC_NS = 16
SC_NW = SC_NC * SC_NS
SC_BPW = N_OBJ // SC_NW


def _pad2(x, r, c):
    return jnp.pad(x, ((0, r - x.shape[0]), (0, c - x.shape[1])))


def _gate_pack_T(W):
    blocks = [_pad2(W[g * D_W:(g + 1) * D_W, :].T, DP, DP) for g in range(3)]
    return jnp.concatenate(blocks, axis=1)


def _gate_pack_bias(b):
    segs = [jnp.pad(b[g * D_W:(g + 1) * D_W], (0, DP - D_W)) for g in range(3)]
    return jnp.concatenate(segs).reshape(1, G3)


def _seg_pack_T(W, nseg):
    dout = W.shape[0]
    blocks = [_pad2(W[:, g * D_W:(g + 1) * D_W].T, DP, dout) for g in range(nseg)]
    return jnp.concatenate(blocks, axis=0)


def _gru_argmax_body(seq_ref, lens_ref, wif_ref, whf_ref, bif_ref, bhf_ref,
                     wib_ref, whb_ref, bib_ref, bhb_ref, cls_ref,
                     h_ref, idx_ref):
    lens = lens_ref[...]
    bif = bif_ref[...]
    bhf = bhf_ref[...]
    bib = bib_ref[...]
    bhb = bhb_ref[...]
    wif = wif_ref[...]
    whf = whf_ref[...]
    wib = wib_ref[...]
    whb = whb_ref[...]

    def gstep(x, h, wi, wh, bi, bh):
        gi = jnp.dot(x, wi, preferred_element_type=jnp.float32) + bi
        gh = jnp.dot(h, wh, preferred_element_type=jnp.float32) + bh
        r = jax.nn.sigmoid(gi[:, :DP] + gh[:, :DP])
        z = jax.nn.sigmoid(gi[:, DP:2 * DP] + gh[:, DP:2 * DP])
        n = jnp.tanh(gi[:, 2 * DP:] + r * gh[:, 2 * DP:])
        return (1.0 - z) * n + z * h

    h_f = jnp.zeros((RP, DP), jnp.float32)
    h_b = jnp.zeros((RP, DP), jnp.float32)
    for t in range(MAX_LEN):
        tb = MAX_LEN - 1 - t
        hf_n = gstep(seq_ref[t], h_f, wif, whf, bif, bhf)
        hb_n = gstep(seq_ref[tb], h_b, wib, whb, bib, bhb)
        h_f = jnp.where(lens > t, hf_n, h_f)
        h_b = jnp.where(lens > tb, hb_n, h_b)
    h_ref[...] = h_f + h_b

    s = cls_ref[...]
    m = jnp.max(s, axis=1, keepdims=True)
    io = lax.broadcasted_iota(jnp.int32, s.shape, 1)
    idx_ref[...] = jnp.min(jnp.where(s >= m, io, N_CLS), axis=1, keepdims=True)


def _attn_body(pooled_ref, fall_ref, wq_ref, bq_ref, w1_ref, b1_ref, w2_ref,
               wr_ref, ur_ref, br_ref, wc_ref, uc_ref, bc_ref,
               wm_ref, bm_ref, wu_ref, bu_ref, out_ref):
    pooled = pooled_ref[...]
    w2 = w2_ref[...]
    ur = ur_ref[...]
    uc = uc_ref[...]
    br = br_ref[...]
    bc = bc_ref[...]

    f = [fall_ref[:, k * DP:(k + 1) * DP] for k in range(TOP_K)]
    validc = fall_ref[:, TOP_K * DP:TOP_K * DP + 1]

    q = jnp.tanh(jnp.dot(pooled, wq_ref[...],
                         preferred_element_type=jnp.float32) + bq_ref[...])

    fq = [f[k] * q for k in range(TOP_K)]
    afq = [jnp.abs(f[k] - q) for k in range(TOP_K)]
    fwr = [jnp.dot(f[k], wr_ref[...], preferred_element_type=jnp.float32) + br
           for k in range(TOP_K)]
    fwc = [jnp.dot(f[k], wc_ref[...], preferred_element_type=jnp.float32) + bc
           for k in range(TOP_K)]

    m = q
    for _ in range(T_M):
        logit = []
        for k in range(TOP_K):
            z = jnp.concatenate([fq[k], f[k] * m, afq[k], jnp.abs(f[k] - m)],
                                axis=1)
            h1 = jnp.tanh(jnp.dot(z, w1_ref[...],
                                  preferred_element_type=jnp.float32) + b1_ref[...])
            logit.append(jnp.sum(h1 * w2, axis=1, keepdims=True))
        lmax = logit[0]
        for k in range(1, TOP_K):
            lmax = jnp.maximum(lmax, logit[k])
        e = [jnp.exp(logit[k] - lmax) for k in range(TOP_K)]
        esum = e[0]
        for k in range(1, TOP_K):
            esum = esum + e[k]
        rinv = 1.0 / esum
        h = jnp.zeros((N_OBJ, DP), jnp.float32)
        for k in range(TOP_K):
            g = e[k] * rinv
            r = jax.nn.sigmoid(fwr[k] + jnp.dot(h, ur,
                                                preferred_element_type=jnp.float32))
            ht = jnp.tanh(fwc[k] + jnp.dot(r * h, uc,
                                           preferred_element_type=jnp.float32))
            h = g * ht + (1.0 - g) * h
        mcat = jnp.concatenate([m, h, q], axis=1)
        m = jax.nn.relu(jnp.dot(mcat, wm_ref[...],
                                preferred_element_type=jnp.float32) + bm_ref[...])

    ucat = jnp.concatenate([pooled, m], axis=1)
    upd = jax.nn.relu(jnp.dot(ucat, wu_ref[...],
                              preferred_element_type=jnp.float32) + bu_ref[...])
    out_ref[...] = jnp.where(validc > 0.5, upd, pooled)


def _sc_gather_body(table_hbm, idx_hbm, out_hbm, idx_v, rows_v, sem):
    wid = lax.axis_index("s") * SC_NC + lax.axis_index("c")
    base = wid * SC_BPW
    pltpu.sync_copy(idx_hbm.at[pl.ds(base, SC_BPW)], idx_v)
    pltpu.async_copy(table_hbm.at[idx_v], rows_v, sem).wait()
    pltpu.sync_copy(rows_v, out_hbm.at[pl.ds(base, SC_BPW)])


@functools.cache
def _sc_gather():
    return pl.kernel(
        _sc_gather_body,
        out_type=jax.ShapeDtypeStruct((N_OBJ, D_TAB), jnp.float32),
        mesh=plsc.VectorSubcoreMesh(core_axis_name="c", subcore_axis_name="s"),
        scratch_types=[
            pltpu.VMEM((SC_BPW,), jnp.int32),
            pltpu.VMEM((SC_BPW, D_TAB), jnp.float32),
            pltpu.SemaphoreType.DMA,
        ],
    )


def kernel(pooled_object_features, cls_score_object, concept_tokens,
           concept_lengths, valid_class_mask,
           Wi_f, Wh_f, bi_f, bh_f, Wi_b, Wh_b, bi_b, bh_b,
           Wq_w, Wq_b, W1_w, W1_b, W2_w, W2_b,
           Wr, Ur, br, Wc, Uc, bc, Wm_w, Wm_b, Wu_w, Wu_b):
    f32 = jnp.float32

    seq = concept_tokens.reshape(ROWS, MAX_LEN, D_W).transpose(1, 0, 2)
    seq = jnp.pad(seq, ((0, 0), (0, RP - ROWS), (0, DP - D_W)))
    lens = jnp.pad(concept_lengths.reshape(ROWS), (0, RP - ROWS)).reshape(RP, 1)

    wif = _gate_pack_T(Wi_f)
    whf = _gate_pack_T(Wh_f)
    wib = _gate_pack_T(Wi_b)
    whb = _gate_pack_T(Wh_b)
    bif = _gate_pack_bias(bi_f)
    bhf = _gate_pack_bias(bh_f)
    bib = _gate_pack_bias(bi_b)
    bhb = _gate_pack_bias(bh_b)

    h_tab, idx = pl.pallas_call(
        _gru_argmax_body,
        out_shape=(jax.ShapeDtypeStruct((RP, DP), f32),
                   jax.ShapeDtypeStruct((N_OBJ, 1), jnp.int32)),
    )(seq, lens, wif, whf, bif, bhf, wib, whb, bib, bhb, cls_score_object)

    tab = h_tab[:ROWS].reshape(N_CLS, TOP_K * DP)
    vcol = valid_class_mask.astype(f32).reshape(N_CLS, 1)
    tab = jnp.concatenate(
        [tab, vcol, jnp.zeros((N_CLS, D_TAB - TOP_K * DP - 1), f32)], axis=1)

    f_all = _sc_gather()(tab, idx.reshape(N_OBJ))

    wq = _pad2(Wq_w.T, D_OBJ, DP)
    bq = jnp.pad(Wq_b, (0, DP - D_W)).reshape(1, DP)
    w1 = _seg_pack_T(W1_w, 4)
    b1 = W1_b.reshape(1, 512)
    w2 = W2_w.reshape(1, 512)
    wr = _pad2(Wr.T, DP, DP)
    urp = _pad2(Ur.T, DP, DP)
    wcp = _pad2(Wc.T, DP, DP)
    ucp = _pad2(Uc.T, DP, DP)
    brp = jnp.pad(br, (0, DP - D_W)).reshape(1, DP)
    bcp = jnp.pad(bc, (0, DP - D_W)).reshape(1, DP)
    wm = _seg_pack_T(jnp.pad(Wm_w, ((0, DP - D_W), (0, 0))), 3)
    bm = jnp.pad(Wm_b, (0, DP - D_W)).reshape(1, DP)
    wu = jnp.concatenate([Wu_w[:, :D_OBJ].T,
                          _pad2(Wu_w[:, D_OBJ:].T, DP, D_OBJ)], axis=0)
    bu = Wu_b.reshape(1, D_OBJ)

    out = pl.pallas_call(
        _attn_body,
        out_shape=jax.ShapeDtypeStruct((N_OBJ, D_OBJ), f32),
    )(pooled_object_features, f_all, wq, bq, w1, b1, w2,
      wr, urp, brp, wcp, ucp, bcp, wm, bm, wu, bu)
    return out


# speedup vs baseline: 6.1839x; 1.0410x over previous
import functools

import jax
import jax.numpy as jnp
from jax import lax
from jax.experimental import pallas as pl
from jax.experimental.pallas import tpu as pltpu
from jax.experimental.pallas import tpu_sc as plsc

N_OBJ = 512
N_CLS = 151
TOP_K = 5
MAX_LEN = 16
D_W = 300
DP = 384
D_OBJ = 512
T_M = 3
ROWS = N_CLS * TOP_K
RP = 768
G3 = 3 * DP
D_TAB = TOP_K * DP + 128

SC_NC = 2
SC_NS = 16
SC_NW = SC_NC * SC_NS
SC_BPW = N_OBJ // SC_NW


def _pad2(x, r, c):
    return jnp.pad(x, ((0, r - x.shape[0]), (0, c - x.shape[1])))


def _gate_pack_T(W):
    blocks = [_pad2(W[g * D_W:(g + 1) * D_W, :].T, DP, DP) for g in range(3)]
    return jnp.concatenate(blocks, axis=1)


def _gate_pack_bias(b):
    segs = [jnp.pad(b[g * D_W:(g + 1) * D_W], (0, DP - D_W)) for g in range(3)]
    return jnp.concatenate(segs).reshape(1, G3)


def _seg_pack_T(W, nseg):
    dout = W.shape[0]
    blocks = [_pad2(W[:, g * D_W:(g + 1) * D_W].T, DP, dout) for g in range(nseg)]
    return jnp.concatenate(blocks, axis=0)


def _gru_argmax_body(seq_ref, lens_ref, wif_ref, whf_ref, bif_ref, bhf_ref,
                     wib_ref, whb_ref, bib_ref, bhb_ref, cls_ref,
                     h_ref, idx_ref):
    lens = lens_ref[...]
    bif = bif_ref[...]
    bhf = bhf_ref[...]
    bib = bib_ref[...]
    bhb = bhb_ref[...]
    wif = wif_ref[...]
    whf = whf_ref[...]
    wib = wib_ref[...]
    whb = whb_ref[...]

    def gstep(x, h, wi, wh, bi, bh):
        gi = jnp.dot(x, wi, preferred_element_type=jnp.float32) + bi
        gh = jnp.dot(h.astype(jnp.bfloat16), wh,
                     preferred_element_type=jnp.float32) + bh
        r = jax.nn.sigmoid(gi[:, :DP] + gh[:, :DP])
        z = jax.nn.sigmoid(gi[:, DP:2 * DP] + gh[:, DP:2 * DP])
        n = jnp.tanh(gi[:, 2 * DP:] + r * gh[:, 2 * DP:])
        return (1.0 - z) * n + z * h

    h_f = jnp.zeros((RP, DP), jnp.float32)
    h_b = jnp.zeros((RP, DP), jnp.float32)
    for t in range(MAX_LEN):
        tb = MAX_LEN - 1 - t
        hf_n = gstep(seq_ref[t], h_f, wif, whf, bif, bhf)
        hb_n = gstep(seq_ref[tb], h_b, wib, whb, bib, bhb)
        h_f = jnp.where(lens > t, hf_n, h_f)
        h_b = jnp.where(lens > tb, hb_n, h_b)
    h_ref[...] = h_f + h_b

    s = cls_ref[...]
    m = jnp.max(s, axis=1, keepdims=True)
    io = lax.broadcasted_iota(jnp.int32, s.shape, 1)
    idx_ref[...] = jnp.min(jnp.where(s >= m, io, N_CLS), axis=1, keepdims=True)


def _attn_body(pooled_ref, fall_ref, wq_ref, bq_ref, w1_ref, b1_ref, w2_ref,
               wr_ref, ur_ref, br_ref, wc_ref, uc_ref, bc_ref,
               wm_ref, bm_ref, wu_ref, bu_ref, out_ref):
    bf16 = jnp.bfloat16

    def dot(a, b):
        return jnp.dot(a.astype(bf16), b, preferred_element_type=jnp.float32)

    pooled = pooled_ref[...]
    w2 = w2_ref[...]
    ur = ur_ref[...]
    uc = uc_ref[...]
    br = br_ref[...]
    bc = bc_ref[...]

    f = [fall_ref[:, k * DP:(k + 1) * DP] for k in range(TOP_K)]
    validc = fall_ref[:, TOP_K * DP:TOP_K * DP + 1]

    q = jnp.tanh(dot(pooled, wq_ref[...]) + bq_ref[...])

    fq = [f[k] * q for k in range(TOP_K)]
    afq = [jnp.abs(f[k] - q) for k in range(TOP_K)]
    fwr = [dot(f[k], wr_ref[...]) + br for k in range(TOP_K)]
    fwc = [dot(f[k], wc_ref[...]) + bc for k in range(TOP_K)]

    m = q
    for _ in range(T_M):
        logit = []
        for k in range(TOP_K):
            z = jnp.concatenate([fq[k], f[k] * m, afq[k], jnp.abs(f[k] - m)],
                                axis=1)
            h1 = jnp.tanh(dot(z, w1_ref[...]) + b1_ref[...])
            logit.append(jnp.sum(h1 * w2, axis=1, keepdims=True))
        lmax = logit[0]
        for k in range(1, TOP_K):
            lmax = jnp.maximum(lmax, logit[k])
        e = [jnp.exp(logit[k] - lmax) for k in range(TOP_K)]
        esum = e[0]
        for k in range(1, TOP_K):
            esum = esum + e[k]
        rinv = 1.0 / esum
        h = jnp.zeros((N_OBJ, DP), jnp.float32)
        for k in range(TOP_K):
            g = e[k] * rinv
            r = jax.nn.sigmoid(fwr[k] + dot(h, ur))
            ht = jnp.tanh(fwc[k] + dot(r * h, uc))
            h = g * ht + (1.0 - g) * h
        mcat = jnp.concatenate([m, h, q], axis=1)
        m = jax.nn.relu(dot(mcat, wm_ref[...]) + bm_ref[...])

    ucat = jnp.concatenate([pooled, m], axis=1)
    upd = jax.nn.relu(dot(ucat, wu_ref[...]) + bu_ref[...])
    out_ref[...] = jnp.where(validc > 0.5, upd, pooled)


def _sc_gather_body(table_hbm, idx_hbm, out_hbm, idx_v, rows_v, sem):
    wid = lax.axis_index("s") * SC_NC + lax.axis_index("c")
    base = wid * SC_BPW
    pltpu.sync_copy(idx_hbm.at[pl.ds(base, SC_BPW)], idx_v)
    pltpu.async_copy(table_hbm.at[idx_v], rows_v, sem).wait()
    pltpu.sync_copy(rows_v, out_hbm.at[pl.ds(base, SC_BPW)])


@functools.cache
def _sc_gather():
    return pl.kernel(
        _sc_gather_body,
        out_type=jax.ShapeDtypeStruct((N_OBJ, D_TAB), jnp.float32),
        mesh=plsc.VectorSubcoreMesh(core_axis_name="c", subcore_axis_name="s"),
        scratch_types=[
            pltpu.VMEM((SC_BPW,), jnp.int32),
            pltpu.VMEM((SC_BPW, D_TAB), jnp.float32),
            pltpu.SemaphoreType.DMA,
        ],
    )


def _prep_gru_inputs(concept_tokens, concept_lengths,
                     Wi_f, Wh_f, bi_f, bh_f, Wi_b, Wh_b, bi_b, bh_b):
    bf16 = jnp.bfloat16
    seq = concept_tokens.reshape(ROWS, MAX_LEN, D_W).transpose(1, 0, 2)
    seq = jnp.pad(seq, ((0, 0), (0, RP - ROWS), (0, DP - D_W))).astype(bf16)
    lens = jnp.pad(concept_lengths.reshape(ROWS), (0, RP - ROWS)).reshape(RP, 1)
    return (seq, lens,
            _gate_pack_T(Wi_f).astype(bf16), _gate_pack_T(Wh_f).astype(bf16),
            _gate_pack_bias(bi_f), _gate_pack_bias(bh_f),
            _gate_pack_T(Wi_b).astype(bf16), _gate_pack_T(Wh_b).astype(bf16),
            _gate_pack_bias(bi_b), _gate_pack_bias(bh_b))


def _prep_attn_weights(Wq_w, Wq_b, W1_w, W1_b, W2_w,
                       Wr, Ur, br, Wc, Uc, bc, Wm_w, Wm_b, Wu_w, Wu_b):
    bf16 = jnp.bfloat16
    wq = _pad2(Wq_w.T, D_OBJ, DP).astype(bf16)
    bq = jnp.pad(Wq_b, (0, DP - D_W)).reshape(1, DP)
    w1 = _seg_pack_T(W1_w, 4).astype(bf16)
    b1 = W1_b.reshape(1, 512)
    w2 = W2_w.reshape(1, 512)
    wr = _pad2(Wr.T, DP, DP).astype(bf16)
    urp = _pad2(Ur.T, DP, DP).astype(bf16)
    wcp = _pad2(Wc.T, DP, DP).astype(bf16)
    ucp = _pad2(Uc.T, DP, DP).astype(bf16)
    brp = jnp.pad(br, (0, DP - D_W)).reshape(1, DP)
    bcp = jnp.pad(bc, (0, DP - D_W)).reshape(1, DP)
    wm = _seg_pack_T(jnp.pad(Wm_w, ((0, DP - D_W), (0, 0))), 3).astype(bf16)
    bm = jnp.pad(Wm_b, (0, DP - D_W)).reshape(1, DP)
    wu = jnp.concatenate([Wu_w[:, :D_OBJ].T,
                          _pad2(Wu_w[:, D_OBJ:].T, DP, D_OBJ)],
                         axis=0).astype(bf16)
    bu = Wu_b.reshape(1, D_OBJ)
    return (wq, bq, w1, b1, w2, wr, urp, brp, wcp, ucp, bcp, wm, bm, wu, bu)


def _pack_table(h_tab, valid_class_mask):
    f32 = jnp.float32
    tab = h_tab[:ROWS].reshape(N_CLS, TOP_K * DP)
    vcol = valid_class_mask.astype(f32).reshape(N_CLS, 1)
    return jnp.concatenate(
        [tab, vcol, jnp.zeros((N_CLS, D_TAB - TOP_K * DP - 1), f32)], axis=1)


def kernel(pooled_object_features, cls_score_object, concept_tokens,
           concept_lengths, valid_class_mask,
           Wi_f, Wh_f, bi_f, bh_f, Wi_b, Wh_b, bi_b, bh_b,
           Wq_w, Wq_b, W1_w, W1_b, W2_w, W2_b,
           Wr, Ur, br, Wc, Uc, bc, Wm_w, Wm_b, Wu_w, Wu_b):
    f32 = jnp.float32

    gru_in = _prep_gru_inputs(concept_tokens, concept_lengths,
                              Wi_f, Wh_f, bi_f, bh_f, Wi_b, Wh_b, bi_b, bh_b)

    h_tab, idx = pl.pallas_call(
        _gru_argmax_body,
        out_shape=(jax.ShapeDtypeStruct((RP, DP), f32),
                   jax.ShapeDtypeStruct((N_OBJ, 1), jnp.int32)),
    )(*gru_in, cls_score_object)

    tab = _pack_table(h_tab, valid_class_mask)
    f_all = _sc_gather()(tab, idx.reshape(N_OBJ))

    aw = _prep_attn_weights(Wq_w, Wq_b, W1_w, W1_b, W2_w,
                            Wr, Ur, br, Wc, Uc, bc, Wm_w, Wm_b, Wu_w, Wu_b)
    out = pl.pallas_call(
        _attn_body,
        out_shape=jax.ShapeDtypeStruct((N_OBJ, D_OBJ), f32),
    )(pooled_object_features, f_all, *aw)
    return out
